# Initial kernel scaffold; baseline (speedup 1.0000x reference)
import jax
import jax.numpy as jnp
from jax.experimental import pallas as pl


def kernel(prop_bond_aromatic, prop_bond_conjugated, prop_bond_stereo, aromatic_table, conjugated_table, stereo_table):
    raise NotImplementedError("write your pallas kernel here")


# speedup vs baseline: 7.4195x; 7.4195x over previous
import functools

import jax
import jax.numpy as jnp
from jax import lax
from jax.experimental import pallas as pl
from jax.experimental.pallas import tpu as pltpu
from jax.experimental.pallas import tpu_sc as plsc

B, L, D = 4096, 200, 128
N = B * L
NA, NC, NS = 3, 3, 7
NCOMBO = NA * NC * NS


def _combo_body(a_ref, c_ref, s_ref, o_ref):
    r = lax.broadcasted_iota(jnp.int32, (NCOMBO, D), 0)
    ia = r // (NC * NS)
    ic = (r // NS) % NC
    ik = r % NS
    acc = jnp.zeros((NCOMBO, D), jnp.float32)
    for i in range(1, NA):
        acc = acc + jnp.where(ia == i, 1.0, 0.0) * a_ref[i, :]
    for j in range(1, NC):
        acc = acc + jnp.where(ic == j, 1.0, 0.0) * c_ref[j, :]
    for k in range(1, NS):
        acc = acc + jnp.where(ik == k, 1.0, 0.0) * s_ref[k, :]
    o_ref[...] = acc


_combo_call = pl.pallas_call(
    _combo_body,
    out_shape=jax.ShapeDtypeStruct((NCOMBO, D), jnp.float32),
)


_NCORES = 2
_NSUB = 16
_NW = _NCORES * _NSUB
_LANES = 16
CHUNK = 128
ROWS_PW = N // _NW
NCHUNK = ROWS_PW // CHUNK

@functools.lru_cache(maxsize=1)
def _get_sc_embed():
    mesh = plsc.VectorSubcoreMesh(core_axis_name="c", subcore_axis_name="s")

    @functools.partial(
        pl.kernel,
        mesh=mesh,
        out_type=jax.ShapeDtypeStruct((N, D), jnp.float32),
        scratch_types=[
            pltpu.VMEM((CHUNK,), jnp.int32),
            pltpu.VMEM((CHUNK,), jnp.int32),
            pltpu.VMEM((CHUNK,), jnp.int32),
            pltpu.VMEM((CHUNK,), jnp.int32),
            pltpu.VMEM((CHUNK, D), jnp.float32),
            pltpu.SemaphoreType.DMA,
        ],
    )
    def _sc_embed(ia_hbm, ic_hbm, is_hbm, combo_hbm, out_hbm,
                  ia_v, ic_v, is_v, fx_v, rows_v, sem):
        wid = lax.axis_index("s") * _NCORES + lax.axis_index("c")
        w_base = wid * ROWS_PW

        def step(g, carry):
            base = w_base + g * CHUNK
            pltpu.sync_copy(ia_hbm.at[pl.ds(base, CHUNK)], ia_v)
            pltpu.sync_copy(ic_hbm.at[pl.ds(base, CHUNK)], ic_v)
            pltpu.sync_copy(is_hbm.at[pl.ds(base, CHUNK)], is_v)
            for i in range(CHUNK // _LANES):
                sl = pl.ds(i * _LANES, _LANES)
                fx_v[sl] = ia_v[sl] * (NC * NS) + ic_v[sl] * NS + is_v[sl]
            pltpu.async_copy(combo_hbm.at[fx_v], rows_v, sem).wait()
            pltpu.sync_copy(rows_v, out_hbm.at[pl.ds(base, CHUNK)])
            return carry

        lax.fori_loop(0, NCHUNK, step, 0)

    return _sc_embed


def kernel(prop_bond_aromatic, prop_bond_conjugated, prop_bond_stereo,
           aromatic_table, conjugated_table, stereo_table):
    combo = _combo_call(aromatic_table, conjugated_table, stereo_table)
    ia = prop_bond_aromatic.reshape(N).astype(jnp.int32)
    ic = prop_bond_conjugated.reshape(N).astype(jnp.int32)
    ik = prop_bond_stereo.reshape(N).astype(jnp.int32)
    out = _get_sc_embed()(ia, ic, ik, combo)
    return out.reshape(B, L, D)


# speedup vs baseline: 8.1811x; 1.1026x over previous
import functools

import jax
import jax.numpy as jnp
from jax import lax
from jax.experimental import pallas as pl
from jax.experimental.pallas import tpu as pltpu
from jax.experimental.pallas import tpu_sc as plsc

B, L, D = 4096, 200, 128
N = B * L
NA, NC, NS = 3, 3, 7
NCOMBO = NA * NC * NS


def _combo_body(a_ref, c_ref, s_ref, o_ref):
    r = lax.broadcasted_iota(jnp.int32, (NCOMBO, D), 0)
    ia = r // (NC * NS)
    ic = (r // NS) % NC
    ik = r % NS
    acc = jnp.zeros((NCOMBO, D), jnp.float32)
    for i in range(1, NA):
        acc = acc + jnp.where(ia == i, 1.0, 0.0) * a_ref[i, :]
    for j in range(1, NC):
        acc = acc + jnp.where(ic == j, 1.0, 0.0) * c_ref[j, :]
    for k in range(1, NS):
        acc = acc + jnp.where(ik == k, 1.0, 0.0) * s_ref[k, :]
    o_ref[...] = acc


_combo_call = pl.pallas_call(
    _combo_body,
    out_shape=jax.ShapeDtypeStruct((NCOMBO, D), jnp.float32),
)


_NCORES = 2
_NSUB = 16
_NW = _NCORES * _NSUB
_LANES = 16
CHUNK = 128
ROWS_PW = N // _NW
NCHUNK = ROWS_PW // CHUNK

NBUF = 4
NSUPER = NCHUNK // NBUF


@functools.lru_cache(maxsize=1)
def _get_sc_embed():
    mesh = plsc.VectorSubcoreMesh(core_axis_name="c", subcore_axis_name="s")

    @functools.partial(
        pl.kernel,
        mesh=mesh,
        out_type=jax.ShapeDtypeStruct((N, D), jnp.float32),
        scratch_types=[
            pltpu.VMEM((NBUF, CHUNK), jnp.int32),
            pltpu.VMEM((NBUF, CHUNK), jnp.int32),
            pltpu.VMEM((NBUF, CHUNK), jnp.int32),
            pltpu.VMEM((NBUF, CHUNK), jnp.int32),
            pltpu.VMEM((NBUF, CHUNK, D), jnp.float32),
        ] + [pltpu.SemaphoreType.DMA] * (3 * NBUF),
    )
    def _sc_embed(ia_hbm, ic_hbm, is_hbm, combo_hbm, out_hbm,
                  ia_v, ic_v, is_v, fx_v, rows_v, *sems):
        isem = sems[0:NBUF]
        gsem = sems[NBUF:2 * NBUF]
        ssem = sems[2 * NBUF:3 * NBUF]
        wid = lax.axis_index("s") * _NCORES + lax.axis_index("c")
        w_base = wid * ROWS_PW
        idx_pairs = ((ia_hbm, ia_v), (ic_hbm, ic_v), (is_hbm, is_v))

        def start_idx(g, b):
            base = w_base + g * CHUNK
            for h, v in idx_pairs:
                pltpu.async_copy(h.at[pl.ds(base, CHUNK)], v.at[b], isem[b])

        def wait_idx(b):
            for h, v in idx_pairs:
                pltpu.make_async_copy(h.at[pl.ds(0, CHUNK)], v.at[b],
                                      isem[b]).wait()

        def compute_fused(b):
            for i in range(CHUNK // _LANES):
                sl = pl.ds(i * _LANES, _LANES)
                fx_v[b, sl] = (ia_v[b, sl] * (NC * NS)
                               + ic_v[b, sl] * NS + is_v[b, sl])

        def start_gather(b):
            pltpu.async_copy(combo_hbm.at[fx_v.at[b]], rows_v.at[b], gsem[b])

        def wait_gather(b):
            pltpu.make_async_copy(out_hbm.at[pl.ds(0, CHUNK)], rows_v.at[b],
                                  gsem[b]).wait()

        def start_store(g, b):
            base = w_base + g * CHUNK
            pltpu.async_copy(rows_v.at[b], out_hbm.at[pl.ds(base, CHUNK)],
                             ssem[b])

        def wait_store(b):
            pltpu.make_async_copy(rows_v.at[b], out_hbm.at[pl.ds(0, CHUNK)],
                                  ssem[b]).wait()

        for b in range(NBUF):
            start_idx(b, b)
        for b in range(NBUF):
            wait_idx(b)
            compute_fused(b)
            start_gather(b)
            start_idx(b + NBUF, b)
            if b > 0:
                wait_gather(b - 1)
                start_store(b - 1, b - 1)

        def super_body(it, carry):
            g0 = it * NBUF
            for b in range(NBUF):
                g = g0 + b
                wait_idx(b)
                compute_fused(b)
                wait_store(b)
                start_gather(b)

                @pl.when(g + NBUF < NCHUNK)
                def _():
                    start_idx(g + NBUF, b)

                b1 = (b - 1) % NBUF
                wait_gather(b1)
                start_store(g - 1, b1)
            return carry

        lax.fori_loop(1, NSUPER, super_body, 0)

        last = NCHUNK - 1
        wait_gather(last % NBUF)
        start_store(last, last % NBUF)
        for b in range(NBUF):
            wait_store(b)

    return _sc_embed


def kernel(prop_bond_aromatic, prop_bond_conjugated, prop_bond_stereo,
           aromatic_table, conjugated_table, stereo_table):
    combo = _combo_call(aromatic_table, conjugated_table, stereo_table)
    ia = prop_bond_aromatic.reshape(N).astype(jnp.int32)
    ic = prop_bond_conjugated.reshape(N).astype(jnp.int32)
    ik = prop_bond_stereo.reshape(N).astype(jnp.int32)
    out = _get_sc_embed()(ia, ic, ik, combo)
    return out.reshape(B, L, D)


# speedup vs baseline: 42.6259x; 5.2103x over previous
import functools

import jax
import jax.numpy as jnp
from jax import lax
from jax.experimental import pallas as pl
from jax.experimental.pallas import tpu as pltpu
from jax.experimental.pallas import tpu_sc as plsc

B, L, D = 4096, 200, 128
N = B * L
NA, NC, NS = 3, 3, 7
NCOMBO = 64


def _combo_body(a_ref, c_ref, s_ref, o_ref):
    r = lax.broadcasted_iota(jnp.int32, (NCOMBO, D), 0)
    ia = r // (NC * NS)
    ic = (r // NS) % NC
    ik = r % NS
    acc = jnp.zeros((NCOMBO, D), jnp.float32)
    for i in range(1, NA):
        acc = acc + jnp.where(ia == i, 1.0, 0.0) * a_ref[i, :]
    for j in range(1, NC):
        acc = acc + jnp.where(ic == j, 1.0, 0.0) * c_ref[j, :]
    for k in range(1, NS):
        acc = acc + jnp.where(ik == k, 1.0, 0.0) * s_ref[k, :]
    o_ref[...] = acc


_combo_call = pl.pallas_call(
    _combo_body,
    out_shape=jax.ShapeDtypeStruct((NCOMBO, D), jnp.float32),
)


_NCORES = 2
_NSUB = 16
_NW = _NCORES * _NSUB
_LANES = 16
CHUNK = 128
ROWS_PW = N // _NW
NCHUNK = ROWS_PW // CHUNK

NBUF = 4
NSUPER = NCHUNK // NBUF


@functools.lru_cache(maxsize=1)
def _get_sc_embed():
    mesh = plsc.VectorSubcoreMesh(core_axis_name="c", subcore_axis_name="s")

    @functools.partial(
        pl.kernel,
        mesh=mesh,
        out_type=jax.ShapeDtypeStruct((N, D), jnp.float32),
        scratch_types=[
            pltpu.VMEM((NBUF, CHUNK), jnp.int32),
            pltpu.VMEM((NBUF, CHUNK), jnp.int32),
            pltpu.VMEM((NBUF, CHUNK), jnp.int32),
            pltpu.VMEM((NBUF, CHUNK), jnp.int32),
            pltpu.VMEM((NBUF, CHUNK, D), jnp.float32),
            pltpu.VMEM_SHARED((NCOMBO, D), jnp.float32),
        ] + [pltpu.SemaphoreType.DMA] * (3 * NBUF),
    )
    def _sc_embed(ia_hbm, ic_hbm, is_hbm, combo_hbm, out_hbm,
                  ia_v, ic_v, is_v, fx_v, rows_v, combo_sh, *sems):
        isem = sems[0:NBUF]
        gsem = sems[NBUF:2 * NBUF]
        ssem = sems[2 * NBUF:3 * NBUF]
        wid = lax.axis_index("s") * _NCORES + lax.axis_index("c")
        w_base = wid * ROWS_PW
        idx_pairs = ((ia_hbm, ia_v), (ic_hbm, ic_v), (is_hbm, is_v))

        def start_idx(g, b):
            base = w_base + g * CHUNK
            for h, v in idx_pairs:
                pltpu.async_copy(h.at[pl.ds(base, CHUNK)], v.at[b], isem[b])

        def wait_idx(b):
            for h, v in idx_pairs:
                pltpu.make_async_copy(h.at[pl.ds(0, CHUNK)], v.at[b],
                                      isem[b]).wait()

        def compute_fused(b):
            for i in range(CHUNK // _LANES):
                sl = pl.ds(i * _LANES, _LANES)
                fx_v[b, sl] = (ia_v[b, sl] * (NC * NS)
                               + ic_v[b, sl] * NS + is_v[b, sl])

        def start_gather(b):
            pltpu.async_copy(combo_sh.at[fx_v.at[b]], rows_v.at[b], gsem[b])

        def wait_gather(b):
            pltpu.make_async_copy(out_hbm.at[pl.ds(0, CHUNK)], rows_v.at[b],
                                  gsem[b]).wait()

        def start_store(g, b):
            base = w_base + g * CHUNK
            pltpu.async_copy(rows_v.at[b], out_hbm.at[pl.ds(base, CHUNK)],
                             ssem[b])

        def wait_store(b):
            pltpu.make_async_copy(rows_v.at[b], out_hbm.at[pl.ds(0, CHUNK)],
                                  ssem[b]).wait()

        @pl.when(lax.axis_index("s") == 0)
        def _():
            pltpu.sync_copy(combo_hbm, combo_sh)
        plsc.subcore_barrier()

        for b in range(NBUF):
            start_idx(b, b)
        for b in range(NBUF):
            wait_idx(b)
            compute_fused(b)
            start_gather(b)
            start_idx(b + NBUF, b)
            if b > 0:
                wait_gather(b - 1)
                start_store(b - 1, b - 1)

        def super_body(it, carry):
            g0 = it * NBUF
            for b in range(NBUF):
                g = g0 + b
                wait_idx(b)
                compute_fused(b)
                wait_store(b)
                start_gather(b)

                @pl.when(g + NBUF < NCHUNK)
                def _():
                    start_idx(g + NBUF, b)

                b1 = (b - 1) % NBUF
                wait_gather(b1)
                start_store(g - 1, b1)
            return carry

        lax.fori_loop(1, NSUPER, super_body, 0)

        last = NCHUNK - 1
        wait_gather(last % NBUF)
        start_store(last, last % NBUF)
        for b in range(NBUF):
            wait_store(b)

    return _sc_embed


def kernel(prop_bond_aromatic, prop_bond_conjugated, prop_bond_stereo,
           aromatic_table, conjugated_table, stereo_table):
    combo = _combo_call(aromatic_table, conjugated_table, stereo_table)
    ia = prop_bond_aromatic.reshape(N).astype(jnp.int32)
    ic = prop_bond_conjugated.reshape(N).astype(jnp.int32)
    ik = prop_bond_stereo.reshape(N).astype(jnp.int32)
    out = _get_sc_embed()(ia, ic, ik, combo)
    return out.reshape(B, L, D)
